# baseline (speedup 1.0000x reference)
import functools

import jax
import jax.numpy as jnp
from jax import lax
from jax.experimental import pallas as pl
from jax.experimental.pallas import tpu as pltpu
from jax.experimental.pallas import tpu_sc as plsc

EMB_DIM = 32
PACK = 128 // EMB_DIM
NUM_WORKERS = 32
CHUNK = 128
NSLOT = 2
LA = 1


def _build(batch, fields):
    n_blk = batch // CHUNK
    n_total = fields * n_blk
    assert n_total % NUM_WORKERS == 0
    n_chunk = n_total // NUM_WORKERS
    assert n_chunk % NSLOT == 0 and n_chunk >= 2 * NSLOT

    mesh = plsc.VectorSubcoreMesh(core_axis_name="c", subcore_axis_name="s")

    @functools.partial(
        pl.kernel,
        mesh=mesh,
        out_type=jax.ShapeDtypeStruct((fields, EMB_DIM, batch), jnp.float32),
        scratch_types=(
            [pltpu.VMEM((n_chunk, CHUNK), jnp.int32)]
            + [pltpu.VMEM((NSLOT, CHUNK), jnp.int32)]
            + [pltpu.VMEM((CHUNK, 128), jnp.float32) for _ in range(NSLOT)]
            + [pltpu.VMEM((EMB_DIM, CHUNK), jnp.float32) for _ in range(NSLOT)]
            + [pltpu.SemaphoreType.DMA((NSLOT,)),
               pltpu.SemaphoreType.DMA((NSLOT,))]
        ),
        compiler_params=pltpu.CompilerParams(
            use_tc_tiling_on_sc=True, needs_layout_passes=False),
    )
    def emb(table_hbm, idx_hbm, out_hbm, idx_v, didx_v, *bufs):
        rows = bufs[:NSLOT]
        outs = bufs[NSLOT:2 * NSLOT]
        gsem, osem = bufs[2 * NSLOT], bufs[2 * NSLOT + 1]
        wid = lax.axis_index("s") * 2 + lax.axis_index("c")
        c0 = wid * n_chunk
        pltpu.sync_copy(idx_hbm.at[pl.ds(c0, n_chunk)], idx_v)
        iota16 = lax.iota(jnp.int32, 16)

        def prep(j, slot):
            for kb in range(CHUNK // 16):
                iv = idx_v[j, pl.ds(kb * 16, 16)]
                didx_v[slot, pl.ds(kb * 16, 16)] = lax.shift_right_logical(iv, 2)

        def gstart(slot):
            pltpu.make_async_copy(
                table_hbm.at[didx_v.at[slot]], rows[slot], gsem.at[slot]
            ).start()

        def gwait(slot):
            pltpu.make_async_copy(
                table_hbm.at[didx_v.at[slot]], rows[slot], gsem.at[slot]
            ).wait()

        def extract(j, slot):
            for kb in range(CHUNK // 16):
                iv = idx_v[j, pl.ds(kb * 16, 16)]
                colv = (iv & (PACK - 1)) * EMB_DIM
                rowv = kb * 16 + iota16
                for e in range(EMB_DIM):
                    g = plsc.load_gather(rows[slot], [rowv, colv])
                    outs[slot][e, pl.ds(kb * 16, 16)] = g
                    if e + 1 < EMB_DIM:
                        colv = colv + 1

        def ostart(j, slot):
            c = c0 + j
            f = c // n_blk
            ct = c % n_blk
            pltpu.make_async_copy(
                outs[slot], out_hbm.at[f, :, pl.ds(ct * CHUNK, CHUNK)],
                osem.at[slot],
            ).start()

        def owait(slot):
            pltpu.make_async_copy(
                outs[slot], out_hbm.at[0, :, pl.ds(0, CHUNK)], osem.at[slot]
            ).wait()

        for j in range(LA):
            prep(j, j)
            gstart(j)
        for j in range(NSLOT):
            a = j + LA
            prep(a, a % NSLOT)
            gstart(a % NSLOT)
            gwait(j % NSLOT)
            extract(j, j % NSLOT)
            ostart(j, j % NSLOT)

        def group(gi, carry):
            j0 = gi * NSLOT
            for b in range(NSLOT):
                j = j0 + b
                prep(j + LA, (b + LA) % NSLOT)
                gstart((b + LA) % NSLOT)
                gwait(b)
                owait(b)
                extract(j, b)
                ostart(j, b)
            return carry

        lax.fori_loop(1, n_chunk // NSLOT - 1, group, 0)

        for j in range(n_chunk - NSLOT, n_chunk):
            a = j + LA
            if a < n_chunk:
                prep(a, a % NSLOT)
                gstart(a % NSLOT)
            gwait(j % NSLOT)
            owait(j % NSLOT)
            extract(j, j % NSLOT)
            ostart(j, j % NSLOT)
        for s in range(NSLOT):
            owait(s)

    return emb


def kernel(x, weight):
    batch, fields = x.shape
    dict_size = weight.shape[0]
    table = weight.reshape(dict_size // PACK, 128)
    idx = x.T.reshape(fields * (batch // CHUNK), CHUNK)
    out = _build(batch, fields)(table, idx)
    return out.transpose(2, 0, 1)


# speedup vs baseline: 1.3508x; 1.3508x over previous
import functools

import jax
import jax.numpy as jnp
from jax import lax
from jax.experimental import pallas as pl
from jax.experimental.pallas import tpu as pltpu
from jax.experimental.pallas import tpu_sc as plsc

EMB_DIM = 32
NUM_WORKERS = 32
CHUNK = 128
NSLOT = 2
LA = 1


def _build(batch, fields):
    n_blk = batch // CHUNK
    n_total = fields * n_blk
    assert n_total % NUM_WORKERS == 0
    n_chunk = n_total // NUM_WORKERS
    assert n_chunk % NSLOT == 0 and n_chunk >= 2 * NSLOT

    mesh = plsc.VectorSubcoreMesh(core_axis_name="c", subcore_axis_name="s")

    @functools.partial(
        pl.kernel,
        mesh=mesh,
        out_type=jax.ShapeDtypeStruct((fields, EMB_DIM, batch), jnp.float32),
        scratch_types=(
            [pltpu.VMEM((n_chunk, CHUNK), jnp.int32)]
            + [pltpu.VMEM((CHUNK, 128), jnp.float32) for _ in range(NSLOT)]
            + [pltpu.VMEM((EMB_DIM, CHUNK), jnp.float32) for _ in range(NSLOT)]
            + [pltpu.SemaphoreType.DMA((NSLOT,)),
               pltpu.SemaphoreType.DMA((NSLOT,))]
        ),
        compiler_params=pltpu.CompilerParams(
            use_tc_tiling_on_sc=True, needs_layout_passes=False),
    )
    def emb(table_hbm, idx_hbm, out_hbm, idx_v, *bufs):
        rows = bufs[:NSLOT]
        outs = bufs[NSLOT:2 * NSLOT]
        gsem, osem = bufs[2 * NSLOT], bufs[2 * NSLOT + 1]
        wid = lax.axis_index("s") * 2 + lax.axis_index("c")
        c0 = wid * n_chunk
        pltpu.sync_copy(idx_hbm.at[pl.ds(c0, n_chunk)], idx_v)
        iota16 = lax.iota(jnp.int32, 16)

        def gstart(j, slot):
            pltpu.make_async_copy(
                table_hbm.at[idx_v.at[j]], rows[slot], gsem.at[slot]
            ).start()

        def gwait(j, slot):
            pltpu.make_async_copy(
                table_hbm.at[idx_v.at[j]], rows[slot], gsem.at[slot]
            ).wait()

        def extract(slot):
            def body(kb, carry):
                rowv = kb * 16 + iota16
                colv = iota16
                for e0 in range(EMB_DIM):
                    g = plsc.load_gather(rows[slot], [rowv, colv])
                    plsc.store_scatter(outs[slot], [colv, rowv], g)
                    if e0 + 1 < EMB_DIM:
                        colv = (colv + 1) & (EMB_DIM - 1)
                return carry

            lax.fori_loop(0, CHUNK // 16, body, 0)

        def ostart(j, slot):
            c = c0 + j
            f = c // n_blk
            ct = c % n_blk
            pltpu.make_async_copy(
                outs[slot], out_hbm.at[f, :, pl.ds(ct * CHUNK, CHUNK)],
                osem.at[slot],
            ).start()

        def owait(slot):
            pltpu.make_async_copy(
                outs[slot], out_hbm.at[0, :, pl.ds(0, CHUNK)], osem.at[slot]
            ).wait()

        for j in range(LA):
            gstart(j, j % NSLOT)
        for j in range(NSLOT):
            a = j + LA
            gstart(a, a % NSLOT)
            gwait(j, j % NSLOT)
            extract(j % NSLOT)
            ostart(j, j % NSLOT)

        def group(gi, carry):
            j0 = gi * NSLOT
            for b in range(NSLOT):
                j = j0 + b
                gstart(j + LA, (b + LA) % NSLOT)
                gwait(j, b)
                owait(b)
                extract(b)
                ostart(j, b)
            return carry

        lax.fori_loop(1, n_chunk // NSLOT - 1, group, 0)

        for j in range(n_chunk - NSLOT, n_chunk):
            a = j + LA
            if a < n_chunk:
                gstart(a, a % NSLOT)
            gwait(j, j % NSLOT)
            owait(j % NSLOT)
            extract(j % NSLOT)
            ostart(j, j % NSLOT)
        for s in range(NSLOT):
            owait(s)

    return emb


def kernel(x, weight):
    batch, fields = x.shape
    table = jnp.pad(weight, ((0, 0), (0, 128 - EMB_DIM)))
    idx = x.T.reshape(fields * (batch // CHUNK), CHUNK)
    out = _build(batch, fields)(table, idx)
    return out.transpose(2, 0, 1)


# speedup vs baseline: 2.0263x; 1.5000x over previous
import functools

import jax
import jax.numpy as jnp
from jax import lax
from jax.experimental import pallas as pl
from jax.experimental.pallas import tpu as pltpu
from jax.experimental.pallas import tpu_sc as plsc

EMB_DIM = 32
PACK = 128 // EMB_DIM
NUM_WORKERS = 32
CHUNK = 128
NSLOT = 2
LA = 1
PROWS = 128


def _build_relayout(dict_size):
    n_packed = dict_size // PACK
    UNIT = 32
    units = n_packed // UNIT
    tail_rows = n_packed % UNIT
    base_u = units // NUM_WORKERS
    extra_u = units % NUM_WORKERS
    tail_start = units * UNIT

    mesh = plsc.VectorSubcoreMesh(core_axis_name="c", subcore_axis_name="s")

    @functools.partial(
        pl.kernel,
        mesh=mesh,
        out_type=jax.ShapeDtypeStruct((n_packed, PACK * EMB_DIM), jnp.float32),
        scratch_types=(
            [pltpu.VMEM((EMB_DIM, PACK * PROWS), jnp.float32)
             for _ in range(NSLOT)]
            + [pltpu.VMEM((PROWS, PACK * EMB_DIM), jnp.float32)
               for _ in range(NSLOT)]
            + [pltpu.SemaphoreType.DMA((NSLOT,)),
               pltpu.SemaphoreType.DMA((NSLOT,))]
        ),
        compiler_params=pltpu.CompilerParams(
            use_tc_tiling_on_sc=True, needs_layout_passes=False),
    )
    def relayout(wt_hbm, wt_tail_hbm, out_hbm, *bufs):
        ins = bufs[:NSLOT]
        outs = bufs[NSLOT:2 * NSLOT]
        isem, osem = bufs[2 * NSLOT], bufs[2 * NSLOT + 1]
        wid = lax.axis_index("s") * 2 + lax.axis_index("c")
        count = UNIT * (base_u + jnp.where(wid < extra_u, 1, 0))
        start = UNIT * (base_u * wid + jnp.minimum(wid, extra_u))
        n_win = (UNIT * (base_u + 1) + PROWS - 1) // PROWS
        last = count - PROWS
        iota16 = lax.iota(jnp.int32, 16)

        def wstart(w):
            return start + jnp.minimum(w * PROWS, last)

        def istart(w, slot):
            pltpu.make_async_copy(
                wt_hbm.at[:, pl.ds(wstart(w) * PACK, PACK * PROWS)],
                ins[slot], isem.at[slot],
            ).start()

        def iwait(w, slot):
            pltpu.make_async_copy(
                wt_hbm.at[:, pl.ds(wstart(w) * PACK, PACK * PROWS)],
                ins[slot], isem.at[slot],
            ).wait()

        def transpose(slot):
            def body(i, carry):
                lv = i * 16 + iota16
                pv = lax.shift_right_logical(lv, 2)
                av32 = (lv & (PACK - 1)) * EMB_DIM
                ev = iota16 & (EMB_DIM - 1)
                for e0 in range(EMB_DIM):
                    g = plsc.load_gather(ins[slot], [ev, lv])
                    plsc.store_scatter(outs[slot], [pv, av32 + ev], g)
                    if e0 + 1 < EMB_DIM:
                        ev = (ev + 1) & (EMB_DIM - 1)
                return carry

            lax.fori_loop(0, PACK * PROWS // 16, body, 0)

        def ostart(w, slot):
            pltpu.make_async_copy(
                outs[slot], out_hbm.at[pl.ds(wstart(w), PROWS)], osem.at[slot]
            ).start()

        def owait(w, slot):
            pltpu.make_async_copy(
                outs[slot], out_hbm.at[pl.ds(wstart(w), PROWS)], osem.at[slot]
            ).wait()

        assert n_win % NSLOT == 0 and n_win >= 2 * NSLOT
        istart(0, 0)
        for b in range(NSLOT):
            istart(b + 1, (b + 1) % NSLOT)
            iwait(b, b)
            transpose(b)
            ostart(b, b)

        def group(gi, carry):
            w0 = gi * NSLOT
            for b in range(NSLOT):
                w = w0 + b

                @pl.when(w + 1 < n_win)
                def _():
                    istart(w + 1, (b + 1) % NSLOT)

                iwait(w, b)
                owait(w - NSLOT, b)
                transpose(b)
                ostart(w, b)
            return carry

        lax.fori_loop(1, n_win // NSLOT, group, 0)
        for s in range(NSLOT):
            owait(n_win - NSLOT + s, s)

        if tail_rows:
            @pl.when(wid == NUM_WORKERS - 1)
            def _tail():
                pltpu.sync_copy(wt_tail_hbm, ins[0].at[:, pl.ds(0, 128)])

                def tbody(i, carry):
                    lv = i * 16 + iota16
                    pv = lax.shift_right_logical(lv, 2)
                    av32 = (lv & (PACK - 1)) * EMB_DIM
                    ev = iota16 & (EMB_DIM - 1)
                    for e0 in range(EMB_DIM):
                        g = plsc.load_gather(ins[0], [ev, lv])
                        plsc.store_scatter(outs[0], [pv, av32 + ev], g)
                        if e0 + 1 < EMB_DIM:
                            ev = (ev + 1) & (EMB_DIM - 1)
                    return carry

                lax.fori_loop(0, PACK * tail_rows // 16, tbody, 0)
                pltpu.sync_copy(
                    outs[0].at[pl.ds(0, tail_rows)],
                    out_hbm.at[pl.ds(tail_start, tail_rows)],
                )

    return relayout


def _build_gather(batch, fields, n_packed):
    n_blk = batch // CHUNK
    n_total = fields * n_blk
    assert n_total % NUM_WORKERS == 0
    n_chunk = n_total // NUM_WORKERS
    assert n_chunk % NSLOT == 0 and n_chunk >= 2 * NSLOT

    mesh = plsc.VectorSubcoreMesh(core_axis_name="c", subcore_axis_name="s")

    @functools.partial(
        pl.kernel,
        mesh=mesh,
        out_type=jax.ShapeDtypeStruct((fields, EMB_DIM, batch), jnp.float32),
        scratch_types=(
            [pltpu.VMEM((n_chunk, CHUNK), jnp.int32)]
            + [pltpu.VMEM((NSLOT, CHUNK), jnp.int32)]
            + [pltpu.VMEM((CHUNK, 128), jnp.float32) for _ in range(NSLOT)]
            + [pltpu.VMEM((EMB_DIM, CHUNK), jnp.float32) for _ in range(NSLOT)]
            + [pltpu.SemaphoreType.DMA((NSLOT,)),
               pltpu.SemaphoreType.DMA((NSLOT,))]
        ),
        compiler_params=pltpu.CompilerParams(
            use_tc_tiling_on_sc=True, needs_layout_passes=False),
    )
    def emb(table_hbm, idx_hbm, out_hbm, idx_v, didx_v, *bufs):
        rows = bufs[:NSLOT]
        outs = bufs[NSLOT:2 * NSLOT]
        gsem, osem = bufs[2 * NSLOT], bufs[2 * NSLOT + 1]
        wid = lax.axis_index("s") * 2 + lax.axis_index("c")
        c0 = wid * n_chunk
        pltpu.sync_copy(idx_hbm.at[pl.ds(c0, n_chunk)], idx_v)
        iota16 = lax.iota(jnp.int32, 16)

        def prep(j, slot):
            for kb in range(CHUNK // 16):
                iv = idx_v[j, pl.ds(kb * 16, 16)]
                didx_v[slot, pl.ds(kb * 16, 16)] = lax.shift_right_logical(iv, 2)

        def gstart(slot):
            pltpu.make_async_copy(
                table_hbm.at[didx_v.at[slot]], rows[slot], gsem.at[slot]
            ).start()

        def gwait(slot):
            pltpu.make_async_copy(
                table_hbm.at[didx_v.at[slot]], rows[slot], gsem.at[slot]
            ).wait()

        def extract(j, slot):
            def body(kb, carry):
                rowv = kb * 16 + iota16
                iv = idx_v[j, pl.ds(kb * 16, 16)]
                av32 = (iv & (PACK - 1)) * EMB_DIM
                ev = iota16 & (EMB_DIM - 1)
                for e0 in range(EMB_DIM):
                    g = plsc.load_gather(rows[slot], [rowv, av32 + ev])
                    plsc.store_scatter(outs[slot], [ev, rowv], g)
                    if e0 + 1 < EMB_DIM:
                        ev = (ev + 1) & (EMB_DIM - 1)
                return carry

            lax.fori_loop(0, CHUNK // 16, body, 0)

        def ostart(j, slot):
            c = c0 + j
            f = c // n_blk
            ct = c % n_blk
            pltpu.make_async_copy(
                outs[slot], out_hbm.at[f, :, pl.ds(ct * CHUNK, CHUNK)],
                osem.at[slot],
            ).start()

        def owait(slot):
            pltpu.make_async_copy(
                outs[slot], out_hbm.at[0, :, pl.ds(0, CHUNK)], osem.at[slot]
            ).wait()

        for j in range(LA):
            prep(j, j)
            gstart(j)
        for j in range(NSLOT):
            a = j + LA
            prep(a, a % NSLOT)
            gstart(a % NSLOT)
            gwait(j % NSLOT)
            extract(j, j % NSLOT)
            ostart(j, j % NSLOT)

        def group(gi, carry):
            j0 = gi * NSLOT
            for b in range(NSLOT):
                j = j0 + b
                prep(j + LA, (b + LA) % NSLOT)
                gstart((b + LA) % NSLOT)
                gwait(b)
                owait(b)
                extract(j, b)
                ostart(j, b)
            return carry

        lax.fori_loop(1, n_chunk // NSLOT - 1, group, 0)

        for j in range(n_chunk - NSLOT, n_chunk):
            a = j + LA
            if a < n_chunk:
                prep(a, a % NSLOT)
                gstart(a % NSLOT)
            gwait(j % NSLOT)
            owait(j % NSLOT)
            extract(j, j % NSLOT)
            ostart(j, j % NSLOT)
        for s in range(NSLOT):
            owait(s)

    return emb


def kernel(x, weight):
    batch, fields = x.shape
    dict_size = weight.shape[0]
    n_main = (dict_size // 128) * 128
    wt_tail = jnp.pad(weight[n_main:], ((0, 128 - (dict_size - n_main)), (0, 0))).T
    table = _build_relayout(dict_size)(weight.T, wt_tail)
    idx = x.T.reshape(fields * (batch // CHUNK), CHUNK)
    out = _build_gather(batch, fields, dict_size // PACK)(table, idx)
    return out.transpose(2, 0, 1)
